# Initial kernel scaffold; baseline (speedup 1.0000x reference)
import jax
import jax.numpy as jnp
from jax.experimental import pallas as pl


def kernel(x, z, mask, embed, Wi_f, Wh_f, b_f, Wi_b, Wh_b, b_b, q_pattern, Wq, Wk, Wv, Wo, W_out, b_out):
    raise NotImplementedError("write your pallas kernel here")


# speedup vs baseline: 13.0257x; 13.0257x over previous
import functools

import jax
import jax.numpy as jnp
from jax import lax
from jax.experimental import pallas as pl
from jax.experimental.pallas import tpu as pltpu
from jax.experimental.pallas import tpu_sc as plsc

_B, _T, _V, _E, _HID, _NH = 8, 512, 50000, 256, 256, 8
_ENC = 2 * _HID
_DH = _ENC // _NH
_GH = 4 * _HID
_SCALING = 100.0
_KBUD = int(round(0.20 * _T))


@functools.cache
def _sc_gather_fn():
    info = plsc.get_sparse_core_info()
    nw = info.num_cores * info.num_subcores
    n = _B * _T
    b_per_w = n // nw
    mesh = plsc.VectorSubcoreMesh(core_axis_name="c", subcore_axis_name="s")

    @functools.partial(
        pl.kernel,
        mesh=mesh,
        out_type=jax.ShapeDtypeStruct((n, _E), jnp.float32),
        scratch_types=[
            pltpu.VMEM((b_per_w,), jnp.int32),
            pltpu.VMEM((b_per_w, _E), jnp.float32),
            pltpu.SemaphoreType.DMA,
        ],
    )
    def gather_kernel(table_hbm, idx_hbm, out_hbm, idx_v, rows_v, sem):
        wid = lax.axis_index("s") * info.num_cores + lax.axis_index("c")
        base = wid * b_per_w
        pltpu.sync_copy(idx_hbm.at[pl.ds(base, b_per_w)], idx_v)
        pltpu.async_copy(table_hbm.at[idx_v], rows_v, sem).wait()
        pltpu.sync_copy(rows_v, out_hbm.at[pl.ds(base, b_per_w)])

    return gather_kernel


def _sc_gather(table, idx):
    return _sc_gather_fn()(table, idx)


def _fused_kernel(emb_ref, wif_ref, whf_ref, bf_ref, wib_ref, whb_ref, bb_ref,
                  wqt_ref, qpt_ref, wk_ref, wv_ref, wo_ref, wout_ref, bout_ref,
                  maskf_ref, y_ref, z_ref, gxf_ref, gxb_ref, hseq_ref, vseq_ref):
    f32 = jnp.float32
    emb = emb_ref[...]
    gxf_ref[...] = jnp.dot(emb, wif_ref[...], preferred_element_type=f32) + bf_ref[...]
    gxb_ref[...] = jnp.dot(emb, wib_ref[...], preferred_element_type=f32) + bb_ref[...]

    whf = whf_ref[...]
    whb = whb_ref[...]

    def step(t, carry):
        hf, cf, hb, cb = carry
        gf = gxf_ref[pl.ds(t * _B, _B), :] + jnp.dot(hf, whf, preferred_element_type=f32)
        cf = (jax.nn.sigmoid(gf[:, _HID:2 * _HID]) * cf
              + jax.nn.sigmoid(gf[:, 0:_HID]) * jnp.tanh(gf[:, 2 * _HID:3 * _HID]))
        hf = jax.nn.sigmoid(gf[:, 3 * _HID:4 * _HID]) * jnp.tanh(cf)
        hseq_ref[:, pl.ds(t, 1), 0:_HID] = hf.reshape(_B, 1, _HID)

        tb = _T - 1 - t
        gb = gxb_ref[pl.ds(tb * _B, _B), :] + jnp.dot(hb, whb, preferred_element_type=f32)
        cb = (jax.nn.sigmoid(gb[:, _HID:2 * _HID]) * cb
              + jax.nn.sigmoid(gb[:, 0:_HID]) * jnp.tanh(gb[:, 2 * _HID:3 * _HID]))
        hb = jax.nn.sigmoid(gb[:, 3 * _HID:4 * _HID]) * jnp.tanh(cb)
        hseq_ref[:, pl.ds(tb, 1), _HID:_ENC] = hb.reshape(_B, 1, _HID)
        return hf, cf, hb, cb

    zinit = jnp.zeros((_B, _HID), f32)
    lax.fori_loop(0, _T, step, (zinit, zinit, zinit, zinit))

    qvt = jnp.dot(wqt_ref[...], qpt_ref[...], preferred_element_type=f32)
    erow = lax.broadcasted_iota(jnp.int32, (_ENC, _NH), 0) // _DH
    hcol = lax.broadcasted_iota(jnp.int32, (_ENC, _NH), 1)
    qmat = jnp.where(erow == hcol, qvt, jnp.zeros((), f32))

    wk = wk_ref[...]
    wv = wv_ref[...]
    srows = []
    for b in range(_B):
        h1b = hseq_ref[b]
        kb = jnp.dot(h1b, wk, preferred_element_type=f32)
        vseq_ref[b] = jnp.dot(h1b, wv, preferred_element_type=f32)
        sbt = lax.dot_general(qmat, kb, (((0,), (1,)), ((), ())),
                              preferred_element_type=f32)
        srows.append(_SCALING * sbt)
    scores = jnp.concatenate(srows, axis=0)

    bits = lax.bitcast_convert_type(scores, jnp.int32)
    keys = jnp.where(bits >= 0, bits, jnp.bitwise_xor(bits, jnp.int32(0x7FFFFFFF)))
    cnt0 = jnp.sum((keys >= 0).astype(jnp.int32), axis=1, keepdims=True)
    big = cnt0 >= _KBUD
    lo = jnp.where(big, jnp.int32(0), jnp.int32(-2147483647 - 1))
    hi = jnp.where(big, jnp.int32(2147483647), jnp.int32(-1))

    def bstep(_, lh):
        blo, bhi = lh
        mid = bhi - ((bhi - blo) >> 1)
        cnt = jnp.sum((keys >= mid).astype(jnp.int32), axis=1, keepdims=True)
        ok = cnt >= _KBUD
        return jnp.where(ok, mid, blo), jnp.where(ok, bhi, mid - 1)

    lo, hi = lax.fori_loop(0, 32, bstep, (lo, hi))

    masked = jnp.where(keys >= lo, scores, jnp.full((), -1e9, f32))
    m = jnp.max(masked, axis=1, keepdims=True)
    e = jnp.exp(masked - m)
    probs = e / jnp.sum(e, axis=1, keepdims=True)

    hm = (lax.broadcasted_iota(jnp.int32, (_NH, _ENC), 1) // _DH
          == lax.broadcasted_iota(jnp.int32, (_NH, _ENC), 0)).astype(f32)
    prows = []
    zrows = []
    for b in range(_B):
        pb = probs[b * _NH:(b + 1) * _NH, :]
        pmat = jnp.dot(pb, vseq_ref[b], preferred_element_type=f32)
        prows.append(jnp.sum(pmat * hm, axis=0, keepdims=True))
        zrows.append(jnp.mean(pb, axis=0, keepdims=True))
    pooled = jnp.concatenate(prows, axis=0)
    h2 = jnp.dot(pooled, wo_ref[...], preferred_element_type=f32)
    y_ref[...] = jax.nn.sigmoid(
        jnp.dot(h2, wout_ref[...], preferred_element_type=f32) + bout_ref[...])
    z_ref[...] = jnp.concatenate(zrows, axis=0) * maskf_ref[...]


def _fused_call(emb, wif, whf, bf, wib, whb, bb, wqt, qpt, wk, wv, wo, wout,
                bout, maskf):
    f32 = jnp.float32
    return pl.pallas_call(
        _fused_kernel,
        out_shape=[
            jax.ShapeDtypeStruct((_B, 1), f32),
            jax.ShapeDtypeStruct((_B, _T), f32),
        ],
        scratch_shapes=[
            pltpu.VMEM((_T * _B, _GH), f32),
            pltpu.VMEM((_T * _B, _GH), f32),
            pltpu.VMEM((_B, _T, _ENC), f32),
            pltpu.VMEM((_B, _T, _ENC), f32),
        ],
        compiler_params=pltpu.CompilerParams(
            vmem_limit_bytes=128 * 1024 * 1024,
        ),
    )(emb, wif, whf, bf, wib, whb, bb, wqt, qpt, wk, wv, wo, wout, bout, maskf)


def kernel(x, z, mask, embed, Wi_f, Wh_f, b_f, Wi_b, Wh_b, b_b, q_pattern,
           Wq, Wk, Wv, Wo, W_out, b_out):
    del z
    idx = x.T.reshape(-1)
    emb = _sc_gather(embed, idx)
    y, z_out = _fused_call(
        emb, Wi_f, Wh_f, b_f.reshape(1, _GH), Wi_b, Wh_b, b_b.reshape(1, _GH),
        Wq.T, q_pattern.reshape(_ENC, 1), Wk, Wv, Wo, W_out,
        b_out.reshape(1, 1), mask.astype(jnp.float32))
    return (y, z_out)


# speedup vs baseline: 15.6359x; 1.2004x over previous
import functools

import jax
import jax.numpy as jnp
from jax import lax
from jax.experimental import pallas as pl
from jax.experimental.pallas import tpu as pltpu
from jax.experimental.pallas import tpu_sc as plsc

_B, _T, _V, _E, _HID, _NH = 8, 512, 50000, 256, 256, 8
_ENC = 2 * _HID
_DH = _ENC // _NH
_GH = 4 * _HID
_SCALING = 100.0
_KBUD = int(round(0.20 * _T))


@functools.cache
def _sc_gather_fn():
    info = plsc.get_sparse_core_info()
    nw = info.num_cores * info.num_subcores
    n = _B * _T
    b_per_w = n // nw
    mesh = plsc.VectorSubcoreMesh(core_axis_name="c", subcore_axis_name="s")

    @functools.partial(
        pl.kernel,
        mesh=mesh,
        out_type=jax.ShapeDtypeStruct((n, _E), jnp.float32),
        scratch_types=[
            pltpu.VMEM((b_per_w,), jnp.int32),
            pltpu.VMEM((b_per_w, _E), jnp.float32),
            pltpu.SemaphoreType.DMA,
        ],
    )
    def gather_kernel(table_hbm, idx_hbm, out_hbm, idx_v, rows_v, sem):
        wid = lax.axis_index("s") * info.num_cores + lax.axis_index("c")
        base = wid * b_per_w
        pltpu.sync_copy(idx_hbm.at[pl.ds(base, b_per_w)], idx_v)
        pltpu.async_copy(table_hbm.at[idx_v], rows_v, sem).wait()
        pltpu.sync_copy(rows_v, out_hbm.at[pl.ds(base, b_per_w)])

    return gather_kernel


def _sc_gather(table, idx):
    return _sc_gather_fn()(table, idx)


def _fused_kernel(emb_ref, wif_ref, whf_ref, bf_ref, wib_ref, whb_ref, bb_ref,
                  wqt_ref, qpt_ref, wk_ref, wv_ref, wo_ref, wout_ref, bout_ref,
                  maskf_ref, y_ref, z_ref, gxf_ref, gxb_ref, hf_ref, hb_ref):
    f32 = jnp.float32
    emb = emb_ref[...]
    gxf_ref[...] = jnp.dot(emb, wif_ref[...], preferred_element_type=f32) + bf_ref[...]
    gxb_ref[...] = jnp.dot(emb, wib_ref[...], preferred_element_type=f32) + bb_ref[...]

    whf = whf_ref[...]
    whb = whb_ref[...]

    def substep(t, hf, cf, hb, cb):
        gf = (gxf_ref[pl.ds(t * _B, _B), :]
              + jnp.dot(hf, whf, preferred_element_type=f32))
        cf = (jax.nn.sigmoid(gf[:, _HID:2 * _HID]) * cf
              + jax.nn.sigmoid(gf[:, 0:_HID]) * jnp.tanh(gf[:, 2 * _HID:3 * _HID]))
        hf = jax.nn.sigmoid(gf[:, 3 * _HID:4 * _HID]) * jnp.tanh(cf)
        hf_ref[pl.ds(t * _B, _B), :] = hf

        tb = _T - 1 - t
        gb = (gxb_ref[pl.ds(tb * _B, _B), :]
              + jnp.dot(hb, whb, preferred_element_type=f32))
        cb = (jax.nn.sigmoid(gb[:, _HID:2 * _HID]) * cb
              + jax.nn.sigmoid(gb[:, 0:_HID]) * jnp.tanh(gb[:, 2 * _HID:3 * _HID]))
        hb = jax.nn.sigmoid(gb[:, 3 * _HID:4 * _HID]) * jnp.tanh(cb)
        hb_ref[pl.ds(tb * _B, _B), :] = hb
        return hf, cf, hb, cb

    def step4(i, carry):
        hf, cf, hb, cb = carry
        hf, cf, hb, cb = substep(4 * i, hf, cf, hb, cb)
        hf, cf, hb, cb = substep(4 * i + 1, hf, cf, hb, cb)
        hf, cf, hb, cb = substep(4 * i + 2, hf, cf, hb, cb)
        return substep(4 * i + 3, hf, cf, hb, cb)

    zinit = jnp.zeros((_B, _HID), f32)
    lax.fori_loop(0, _T // 4, step4, (zinit, zinit, zinit, zinit))

    qvt = jnp.dot(wqt_ref[...], qpt_ref[...], preferred_element_type=f32)
    erow = lax.broadcasted_iota(jnp.int32, (_ENC, _NH), 0) // _DH
    hcol = lax.broadcasted_iota(jnp.int32, (_ENC, _NH), 1)
    qmat = jnp.where(erow == hcol, qvt, jnp.zeros((), f32))

    nchunk = 4
    crows = _T * _B // nchunk
    strows = []
    for c in range(nchunk):
        hfc = hf_ref[pl.ds(c * crows, crows), :]
        hbc = hb_ref[pl.ds(c * crows, crows), :]
        ka_c = (jnp.dot(hfc, wk_ref[0:_HID, :], preferred_element_type=f32)
                + jnp.dot(hbc, wk_ref[_HID:_ENC, :], preferred_element_type=f32))
        strows.append(lax.dot_general(qmat, ka_c, (((0,), (1,)), ((), ())),
                                      preferred_element_type=f32))
        gxf_ref[pl.ds(c * crows, crows), 0:_ENC] = (
            jnp.dot(hfc, wv_ref[0:_HID, :], preferred_element_type=f32)
            + jnp.dot(hbc, wv_ref[_HID:_ENC, :], preferred_element_type=f32))
    scorest = jnp.concatenate(strows, axis=1)
    scores2 = _SCALING * jnp.transpose(scorest)

    bits = lax.bitcast_convert_type(scores2, jnp.int32)
    keys2 = jnp.where(bits >= 0, bits, jnp.bitwise_xor(bits, jnp.int32(0x7FFFFFFF)))
    keys = keys2.reshape(_T, _B, _NH)
    cnt0 = jnp.sum((keys >= 0).astype(jnp.int32), axis=0, keepdims=True)
    big = cnt0 >= _KBUD
    lo = jnp.where(big, jnp.int32(0), jnp.int32(-2147483647 - 1))
    hi = jnp.where(big, jnp.int32(2147483647), jnp.int32(-1))

    def bstep(_, lh):
        blo, bhi = lh
        mid = bhi - ((bhi - blo) >> 1)
        cnt = jnp.sum((keys >= mid).astype(jnp.int32), axis=0, keepdims=True)
        ok = cnt >= _KBUD
        return jnp.where(ok, mid, blo), jnp.where(ok, bhi, mid - 1)

    lo, hi = lax.fori_loop(0, 32, bstep, (lo, hi))

    scores3 = scores2.reshape(_T, _B, _NH)
    masked = jnp.where(keys >= lo, scores3, jnp.full((), -1e9, f32))
    m = jnp.max(masked, axis=0, keepdims=True)
    e = jnp.exp(masked - m)
    probs = e / jnp.sum(e, axis=0, keepdims=True)

    z2 = jnp.mean(probs, axis=2)
    z_ref[...] = jnp.transpose(z2) * maskf_ref[...]

    hm = (lax.broadcasted_iota(jnp.int32, (_NH, _ENC), 1) // _DH
          == lax.broadcasted_iota(jnp.int32, (_NH, _ENC), 0)).astype(f32)
    probs2d = probs.reshape(_T * _B, _NH)
    pooled = jnp.zeros((_B, _ENC), f32)
    for c in range(nchunk):
        pexp_c = jnp.dot(probs2d[c * crows:(c + 1) * crows, :], hm,
                         preferred_element_type=f32)
        va_c = gxf_ref[pl.ds(c * crows, crows), 0:_ENC]
        pooled = pooled + jnp.sum(
            (pexp_c * va_c).reshape(crows // _B, _B, _ENC), axis=0)
    h2 = jnp.dot(pooled, wo_ref[...], preferred_element_type=f32)
    y_ref[...] = jax.nn.sigmoid(
        jnp.dot(h2, wout_ref[...], preferred_element_type=f32) + bout_ref[...])


def _fused_call(emb, wif, whf, bf, wib, whb, bb, wqt, qpt, wk, wv, wo, wout,
                bout, maskf):
    f32 = jnp.float32
    return pl.pallas_call(
        _fused_kernel,
        out_shape=[
            jax.ShapeDtypeStruct((_B, 1), f32),
            jax.ShapeDtypeStruct((_B, _T), f32),
        ],
        scratch_shapes=[
            pltpu.VMEM((_T * _B, _GH), f32),
            pltpu.VMEM((_T * _B, _GH), f32),
            pltpu.VMEM((_T * _B, _HID), f32),
            pltpu.VMEM((_T * _B, _HID), f32),
        ],
        compiler_params=pltpu.CompilerParams(
            vmem_limit_bytes=128 * 1024 * 1024,
        ),
    )(emb, wif, whf, bf, wib, whb, bb, wqt, qpt, wk, wv, wo, wout, bout, maskf)


def kernel(x, z, mask, embed, Wi_f, Wh_f, b_f, Wi_b, Wh_b, b_b, q_pattern,
           Wq, Wk, Wv, Wo, W_out, b_out):
    del z
    idx = x.T.reshape(-1)
    emb = _sc_gather(embed, idx)
    y, z_out = _fused_call(
        emb, Wi_f, Wh_f, b_f.reshape(1, _GH), Wi_b, Wh_b, b_b.reshape(1, _GH),
        Wq.T, q_pattern.reshape(_ENC, 1), Wk, Wv, Wo, W_out,
        b_out.reshape(1, 1), mask.astype(jnp.float32))
    return (y, z_out)


# speedup vs baseline: 17.5852x; 1.1247x over previous
import functools

import jax
import jax.numpy as jnp
from jax import lax
from jax.experimental import pallas as pl
from jax.experimental.pallas import tpu as pltpu
from jax.experimental.pallas import tpu_sc as plsc

_B, _T, _V, _E, _HID, _NH = 8, 512, 50000, 256, 256, 8
_ENC = 2 * _HID
_DH = _ENC // _NH
_GH = 4 * _HID
_SCALING = 100.0
_KBUD = int(round(0.20 * _T))


@functools.cache
def _sc_gather_fn():
    info = plsc.get_sparse_core_info()
    nw = info.num_cores * info.num_subcores
    n = _B * _T
    b_per_w = n // nw
    mesh = plsc.VectorSubcoreMesh(core_axis_name="c", subcore_axis_name="s")

    @functools.partial(
        pl.kernel,
        mesh=mesh,
        out_type=jax.ShapeDtypeStruct((n, _E), jnp.float32),
        scratch_types=[
            pltpu.VMEM((b_per_w,), jnp.int32),
            pltpu.VMEM((b_per_w, _E), jnp.float32),
            pltpu.SemaphoreType.DMA,
        ],
    )
    def gather_kernel(table_hbm, idx_hbm, out_hbm, idx_v, rows_v, sem):
        wid = lax.axis_index("s") * info.num_cores + lax.axis_index("c")
        base = wid * b_per_w
        pltpu.sync_copy(idx_hbm.at[pl.ds(base, b_per_w)], idx_v)
        pltpu.async_copy(table_hbm.at[idx_v], rows_v, sem).wait()
        pltpu.sync_copy(rows_v, out_hbm.at[pl.ds(base, b_per_w)])

    return gather_kernel


def _sc_gather(table, idx):
    return _sc_gather_fn()(table, idx)


def _fused_kernel(emb_ref, wif_ref, whf_ref, bf_ref, wib_ref, whb_ref, bb_ref,
                  wqt_ref, qpt_ref, wk_ref, wv_ref, wo_ref, wout_ref, bout_ref,
                  maskf_ref, y_ref, z_ref, gxf_ref, gxb_ref, hseq_ref, vseq_ref):
    f32 = jnp.float32
    emb = emb_ref[...]
    gxf_ref[...] = jnp.dot(emb, wif_ref[...], preferred_element_type=f32) + bf_ref[...]
    gxb_ref[...] = jnp.dot(emb, wib_ref[...], preferred_element_type=f32) + bb_ref[...]

    whf = whf_ref[...]
    whb = whb_ref[...]

    def substep(t, hf, cf, hb, cb):
        gf = (gxf_ref[pl.ds(t * _B, _B), :]
              + jnp.dot(hf, whf, preferred_element_type=f32))
        cf = (jax.nn.sigmoid(gf[:, _HID:2 * _HID]) * cf
              + jax.nn.sigmoid(gf[:, 0:_HID]) * jnp.tanh(gf[:, 2 * _HID:3 * _HID]))
        hf = jax.nn.sigmoid(gf[:, 3 * _HID:4 * _HID]) * jnp.tanh(cf)
        hseq_ref[:, pl.ds(t, 1), 0:_HID] = hf.reshape(_B, 1, _HID)

        tb = _T - 1 - t
        gb = (gxb_ref[pl.ds(tb * _B, _B), :]
              + jnp.dot(hb, whb, preferred_element_type=f32))
        cb = (jax.nn.sigmoid(gb[:, _HID:2 * _HID]) * cb
              + jax.nn.sigmoid(gb[:, 0:_HID]) * jnp.tanh(gb[:, 2 * _HID:3 * _HID]))
        hb = jax.nn.sigmoid(gb[:, 3 * _HID:4 * _HID]) * jnp.tanh(cb)
        hseq_ref[:, pl.ds(tb, 1), _HID:_ENC] = hb.reshape(_B, 1, _HID)
        return hf, cf, hb, cb

    def step8(i, carry):
        carry = (carry[0], carry[1], carry[2], carry[3])
        for j in range(8):
            carry = substep(8 * i + j, *carry)
        return carry

    zinit = jnp.zeros((_B, _HID), f32)
    lax.fori_loop(0, _T // 8, step8, (zinit, zinit, zinit, zinit))

    qvt = jnp.dot(wqt_ref[...], qpt_ref[...], preferred_element_type=f32)
    erow = lax.broadcasted_iota(jnp.int32, (_ENC, _NH), 0) // _DH
    hcol = lax.broadcasted_iota(jnp.int32, (_ENC, _NH), 1)
    qmat = jnp.where(erow == hcol, qvt, jnp.zeros((), f32))

    wk = wk_ref[...]
    wv = wv_ref[...]
    srows = []
    for b in range(_B):
        h1b = hseq_ref[b]
        kb = jnp.dot(h1b, wk, preferred_element_type=f32)
        vseq_ref[b] = jnp.dot(h1b, wv, preferred_element_type=f32)
        sbt = lax.dot_general(qmat, kb, (((0,), (1,)), ((), ())),
                              preferred_element_type=f32)
        srows.append(_SCALING * sbt)
    scores = jnp.concatenate(srows, axis=0)

    bits = lax.bitcast_convert_type(scores, jnp.int32)
    keys = jnp.where(bits >= 0, bits, jnp.bitwise_xor(bits, jnp.int32(0x7FFFFFFF)))
    cnt0 = jnp.sum((keys >= 0).astype(jnp.int32), axis=1, keepdims=True)
    big = cnt0 >= _KBUD
    lo = jnp.where(big, jnp.int32(0), jnp.int32(-2147483647 - 1))
    hi = jnp.where(big, jnp.int32(2147483647), jnp.int32(-1))

    def bstep(_, lh):
        blo, bhi = lh
        mid = bhi - ((bhi - blo) >> 1)
        cnt = jnp.sum((keys >= mid).astype(jnp.int32), axis=1, keepdims=True)
        ok = cnt >= _KBUD
        return jnp.where(ok, mid, blo), jnp.where(ok, bhi, mid - 1)

    lo, hi = lax.fori_loop(0, 32, bstep, (lo, hi))

    masked = jnp.where(keys >= lo, scores, jnp.full((), -1e9, f32))
    m = jnp.max(masked, axis=1, keepdims=True)
    e = jnp.exp(masked - m)
    probs = e / jnp.sum(e, axis=1, keepdims=True)

    hm = (lax.broadcasted_iota(jnp.int32, (_NH, _ENC), 1) // _DH
          == lax.broadcasted_iota(jnp.int32, (_NH, _ENC), 0)).astype(f32)
    prows = []
    zrows = []
    for b in range(_B):
        pb = probs[b * _NH:(b + 1) * _NH, :]
        pmat = jnp.dot(pb, vseq_ref[b], preferred_element_type=f32)
        prows.append(jnp.sum(pmat * hm, axis=0, keepdims=True))
        zrows.append(jnp.mean(pb, axis=0, keepdims=True))
    pooled = jnp.concatenate(prows, axis=0)
    h2 = jnp.dot(pooled, wo_ref[...], preferred_element_type=f32)
    y_ref[...] = jax.nn.sigmoid(
        jnp.dot(h2, wout_ref[...], preferred_element_type=f32) + bout_ref[...])
    z_ref[...] = jnp.concatenate(zrows, axis=0) * maskf_ref[...]


def _fused_call(emb, wif, whf, bf, wib, whb, bb, wqt, qpt, wk, wv, wo, wout,
                bout, maskf):
    f32 = jnp.float32
    return pl.pallas_call(
        _fused_kernel,
        out_shape=[
            jax.ShapeDtypeStruct((_B, 1), f32),
            jax.ShapeDtypeStruct((_B, _T), f32),
        ],
        scratch_shapes=[
            pltpu.VMEM((_T * _B, _GH), f32),
            pltpu.VMEM((_T * _B, _GH), f32),
            pltpu.VMEM((_B, _T, _ENC), f32),
            pltpu.VMEM((_B, _T, _ENC), f32),
        ],
        compiler_params=pltpu.CompilerParams(
            vmem_limit_bytes=128 * 1024 * 1024,
        ),
    )(emb, wif, whf, bf, wib, whb, bb, wqt, qpt, wk, wv, wo, wout, bout, maskf)


def kernel(x, z, mask, embed, Wi_f, Wh_f, b_f, Wi_b, Wh_b, b_b, q_pattern,
           Wq, Wk, Wv, Wo, W_out, b_out):
    del z
    idx = x.T.reshape(-1)
    emb = _sc_gather(embed, idx)
    y, z_out = _fused_call(
        emb, Wi_f, Wh_f, b_f.reshape(1, _GH), Wi_b, Wh_b, b_b.reshape(1, _GH),
        Wq.T, q_pattern.reshape(_ENC, 1), Wk, Wv, Wo, W_out,
        b_out.reshape(1, 1), mask.astype(jnp.float32))
    return (y, z_out)


# speedup vs baseline: 18.3893x; 1.0457x over previous
import functools

import jax
import jax.numpy as jnp
from jax import lax
from jax.experimental import pallas as pl
from jax.experimental.pallas import tpu as pltpu
from jax.experimental.pallas import tpu_sc as plsc

_B, _T, _V, _E, _HID, _NH = 8, 512, 50000, 256, 256, 8
_ENC = 2 * _HID
_DH = _ENC // _NH
_GH = 4 * _HID
_SCALING = 100.0
_KBUD = int(round(0.20 * _T))


@functools.cache
def _sc_gather_fn():
    info = plsc.get_sparse_core_info()
    nw = info.num_cores * info.num_subcores
    n = _B * _T
    b_per_w = n // nw
    mesh = plsc.VectorSubcoreMesh(core_axis_name="c", subcore_axis_name="s")

    @functools.partial(
        pl.kernel,
        mesh=mesh,
        out_type=jax.ShapeDtypeStruct((n, _E), jnp.float32),
        scratch_types=[
            pltpu.VMEM((b_per_w,), jnp.int32),
            pltpu.VMEM((b_per_w, _E), jnp.float32),
            pltpu.SemaphoreType.DMA,
        ],
    )
    def gather_kernel(table_hbm, idx_hbm, out_hbm, idx_v, rows_v, sem):
        wid = lax.axis_index("s") * info.num_cores + lax.axis_index("c")
        base = wid * b_per_w
        pltpu.sync_copy(idx_hbm.at[pl.ds(base, b_per_w)], idx_v)
        pltpu.async_copy(table_hbm.at[idx_v], rows_v, sem).wait()
        pltpu.sync_copy(rows_v, out_hbm.at[pl.ds(base, b_per_w)])

    return gather_kernel


def _sc_gather(table, idx):
    return _sc_gather_fn()(table, idx)


def _fused_kernel(emb_ref, wif_ref, whf_ref, bf_ref, wib_ref, whb_ref, bb_ref,
                  wqt_ref, qpt_ref, wk_ref, wv_ref, wo_ref, wout_ref, bout_ref,
                  maskf_ref, y_ref, z_ref, gxf_ref, gxb_ref, hseq_ref, vseq_ref,
                  wk_v, wv_v, wo_v, sem_k, sem_v, sem_o):
    f32 = jnp.float32
    cp_k = pltpu.make_async_copy(wk_ref, wk_v, sem_k)
    cp_v = pltpu.make_async_copy(wv_ref, wv_v, sem_v)
    cp_o = pltpu.make_async_copy(wo_ref, wo_v, sem_o)
    cp_k.start()
    cp_v.start()
    cp_o.start()
    emb = emb_ref[...]
    gxf_ref[...] = jnp.dot(emb, wif_ref[...], preferred_element_type=f32) + bf_ref[...]
    gxb_ref[...] = jnp.dot(emb, wib_ref[...], preferred_element_type=f32) + bb_ref[...]

    whf = whf_ref[...]
    whb = whb_ref[...]

    def substep(t, hf, cf, hb, cb):
        gf = (gxf_ref[pl.ds(t * _B, _B), :]
              + jnp.dot(hf, whf, preferred_element_type=f32))
        cf = (jax.nn.sigmoid(gf[:, _HID:2 * _HID]) * cf
              + jax.nn.sigmoid(gf[:, 0:_HID]) * jnp.tanh(gf[:, 2 * _HID:3 * _HID]))
        hf = jax.nn.sigmoid(gf[:, 3 * _HID:4 * _HID]) * jnp.tanh(cf)
        hseq_ref[:, pl.ds(t, 1), 0:_HID] = hf.reshape(_B, 1, _HID)

        tb = _T - 1 - t
        gb = (gxb_ref[pl.ds(tb * _B, _B), :]
              + jnp.dot(hb, whb, preferred_element_type=f32))
        cb = (jax.nn.sigmoid(gb[:, _HID:2 * _HID]) * cb
              + jax.nn.sigmoid(gb[:, 0:_HID]) * jnp.tanh(gb[:, 2 * _HID:3 * _HID]))
        hb = jax.nn.sigmoid(gb[:, 3 * _HID:4 * _HID]) * jnp.tanh(cb)
        hseq_ref[:, pl.ds(tb, 1), _HID:_ENC] = hb.reshape(_B, 1, _HID)
        return hf, cf, hb, cb

    def step16(i, carry):
        for j in range(32):
            carry = substep(32 * i + j, *carry)
        return carry

    zinit = jnp.zeros((_B, _HID), f32)
    lax.fori_loop(0, _T // 32, step16, (zinit, zinit, zinit, zinit))

    qvt = jnp.dot(wqt_ref[...], qpt_ref[...], preferred_element_type=f32)
    erow = lax.broadcasted_iota(jnp.int32, (_ENC, _NH), 0) // _DH
    hcol = lax.broadcasted_iota(jnp.int32, (_ENC, _NH), 1)
    qmat = jnp.where(erow == hcol, qvt, jnp.zeros((), f32))

    cp_k.wait()
    cp_v.wait()
    cp_o.wait()
    wk = wk_v[...]
    wv = wv_v[...]
    srows = []
    for b in range(_B):
        h1b = hseq_ref[b]
        kb = jnp.dot(h1b, wk, preferred_element_type=f32)
        vseq_ref[b] = jnp.dot(h1b, wv, preferred_element_type=f32)
        sbt = lax.dot_general(qmat, kb, (((0,), (1,)), ((), ())),
                              preferred_element_type=f32)
        srows.append(_SCALING * sbt)
    scores = jnp.concatenate(srows, axis=0)

    bits = lax.bitcast_convert_type(scores, jnp.int32)
    keys = jnp.where(bits >= 0, bits, jnp.bitwise_xor(bits, jnp.int32(0x7FFFFFFF)))
    cnt0 = jnp.sum((keys >= 0).astype(jnp.int32), axis=1, keepdims=True)
    big = cnt0 >= _KBUD
    lo = jnp.where(big, jnp.int32(0), jnp.int32(-2147483647 - 1))
    hi = jnp.where(big, jnp.int32(2147483647), jnp.int32(-1))

    def bstep(_, lh):
        blo, bhi = lh
        mid = bhi - ((bhi - blo) >> 1)
        cnt = jnp.sum((keys >= mid).astype(jnp.int32), axis=1, keepdims=True)
        ok = cnt >= _KBUD
        return jnp.where(ok, mid, blo), jnp.where(ok, bhi, mid - 1)

    lo, hi = lax.fori_loop(0, 32, bstep, (lo, hi))

    masked = jnp.where(keys >= lo, scores, jnp.full((), -1e9, f32))
    m = jnp.max(masked, axis=1, keepdims=True)
    e = jnp.exp(masked - m)
    probs = e / jnp.sum(e, axis=1, keepdims=True)

    hm = (lax.broadcasted_iota(jnp.int32, (_NH, _ENC), 1) // _DH
          == lax.broadcasted_iota(jnp.int32, (_NH, _ENC), 0)).astype(f32)
    prows = []
    zrows = []
    for b in range(_B):
        pb = probs[b * _NH:(b + 1) * _NH, :]
        pmat = jnp.dot(pb, vseq_ref[b], preferred_element_type=f32)
        prows.append(jnp.sum(pmat * hm, axis=0, keepdims=True))
        zrows.append(jnp.mean(pb, axis=0, keepdims=True))
    pooled = jnp.concatenate(prows, axis=0)
    h2 = jnp.dot(pooled, wo_v[...], preferred_element_type=f32)
    y_ref[...] = jax.nn.sigmoid(
        jnp.dot(h2, wout_ref[...], preferred_element_type=f32) + bout_ref[...])
    z_ref[...] = jnp.concatenate(zrows, axis=0) * maskf_ref[...]


def _fused_call(emb, wif, whf, bf, wib, whb, bb, wqt, qpt, wk, wv, wo, wout,
                bout, maskf):
    f32 = jnp.float32
    return pl.pallas_call(
        _fused_kernel,
        out_shape=[
            jax.ShapeDtypeStruct((_B, 1), f32),
            jax.ShapeDtypeStruct((_B, _T), f32),
        ],
        scratch_shapes=[
            pltpu.VMEM((_T * _B, _GH), f32),
            pltpu.VMEM((_T * _B, _GH), f32),
            pltpu.VMEM((_B, _T, _ENC), f32),
            pltpu.VMEM((_B, _T, _ENC), f32),
            pltpu.VMEM((_ENC, _ENC), f32),
            pltpu.VMEM((_ENC, _ENC), f32),
            pltpu.VMEM((_ENC, _ENC), f32),
            pltpu.SemaphoreType.DMA,
            pltpu.SemaphoreType.DMA,
            pltpu.SemaphoreType.DMA,
        ],
        in_specs=[pl.BlockSpec(memory_space=pltpu.VMEM)] * 9
        + [pl.BlockSpec(memory_space=pl.ANY)] * 3
        + [pl.BlockSpec(memory_space=pltpu.VMEM)] * 3,
        compiler_params=pltpu.CompilerParams(
            vmem_limit_bytes=128 * 1024 * 1024,
        ),
    )(emb, wif, whf, bf, wib, whb, bb, wqt, qpt, wk, wv, wo, wout, bout, maskf)


def kernel(x, z, mask, embed, Wi_f, Wh_f, b_f, Wi_b, Wh_b, b_b, q_pattern,
           Wq, Wk, Wv, Wo, W_out, b_out):
    del z
    idx = x.T.reshape(-1)
    emb = _sc_gather(embed, idx)
    y, z_out = _fused_call(
        emb, Wi_f, Wh_f, b_f.reshape(1, _GH), Wi_b, Wh_b, b_b.reshape(1, _GH),
        Wq.T, q_pattern.reshape(_ENC, 1), Wk, Wv, Wo, W_out,
        b_out.reshape(1, 1), mask.astype(jnp.float32))
    return (y, z_out)


# speedup vs baseline: 18.5204x; 1.0071x over previous
import functools

import jax
import jax.numpy as jnp
from jax import lax
from jax.experimental import pallas as pl
from jax.experimental.pallas import tpu as pltpu
from jax.experimental.pallas import tpu_sc as plsc

_B, _T, _V, _E, _HID, _NH = 8, 512, 50000, 256, 256, 8
_ENC = 2 * _HID
_DH = _ENC // _NH
_GH = 4 * _HID
_SCALING = 100.0
_KBUD = int(round(0.20 * _T))


@functools.cache
def _sc_gather_fn():
    info = plsc.get_sparse_core_info()
    nw = info.num_cores * info.num_subcores
    n = _B * _T
    b_per_w = n // nw
    mesh = plsc.VectorSubcoreMesh(core_axis_name="c", subcore_axis_name="s")

    @functools.partial(
        pl.kernel,
        mesh=mesh,
        out_type=jax.ShapeDtypeStruct((n, _E), jnp.float32),
        scratch_types=[
            pltpu.VMEM((b_per_w,), jnp.int32),
            pltpu.VMEM((b_per_w, _E), jnp.float32),
            pltpu.SemaphoreType.DMA,
        ],
    )
    def gather_kernel(table_hbm, idx_hbm, out_hbm, idx_v, rows_v, sem):
        wid = lax.axis_index("s") * info.num_cores + lax.axis_index("c")
        base = wid * b_per_w
        pltpu.sync_copy(idx_hbm.at[pl.ds(base, b_per_w)], idx_v)
        pltpu.async_copy(table_hbm.at[idx_v], rows_v, sem).wait()
        pltpu.sync_copy(rows_v, out_hbm.at[pl.ds(base, b_per_w)])

    return gather_kernel


def _sc_gather(table, idx):
    return _sc_gather_fn()(table, idx)


def _fused_kernel(emb_ref, wif_ref, whf_ref, bf_ref, wib_ref, whb_ref, bb_ref,
                  wqt_ref, qpt_ref, wk_ref, wv_ref, wo_ref, wout_ref, bout_ref,
                  maskf_ref, y_ref, z_ref, gxf_ref, gxb_ref, hseq_ref, vseq_ref,
                  wk_v, wv_v, wo_v, sem_k, sem_v, sem_o):
    f32 = jnp.float32
    cp_k = pltpu.make_async_copy(wk_ref, wk_v, sem_k)
    cp_v = pltpu.make_async_copy(wv_ref, wv_v, sem_v)
    cp_o = pltpu.make_async_copy(wo_ref, wo_v, sem_o)
    cp_k.start()
    cp_v.start()
    cp_o.start()
    emb = emb_ref[...]
    gxf_ref[...] = jnp.dot(emb, wif_ref[...], preferred_element_type=f32) + bf_ref[...]
    gxb_ref[...] = jnp.dot(emb, wib_ref[...], preferred_element_type=f32) + bb_ref[...]

    whf = whf_ref[...]
    whb = whb_ref[...]

    def substep(t, hf, cf, hb, cb):
        gf = (gxf_ref[pl.ds(t * _B, _B), :]
              + jnp.dot(hf, whf, preferred_element_type=f32))
        cf = (jax.nn.sigmoid(gf[:, _HID:2 * _HID]) * cf
              + jax.nn.sigmoid(gf[:, 0:_HID]) * jnp.tanh(gf[:, 2 * _HID:3 * _HID]))
        hf = jax.nn.sigmoid(gf[:, 3 * _HID:4 * _HID]) * jnp.tanh(cf)
        hseq_ref[:, pl.ds(t, 1), 0:_HID] = hf.reshape(_B, 1, _HID)

        tb = _T - 1 - t
        gb = (gxb_ref[pl.ds(tb * _B, _B), :]
              + jnp.dot(hb, whb, preferred_element_type=f32))
        cb = (jax.nn.sigmoid(gb[:, _HID:2 * _HID]) * cb
              + jax.nn.sigmoid(gb[:, 0:_HID]) * jnp.tanh(gb[:, 2 * _HID:3 * _HID]))
        hb = jax.nn.sigmoid(gb[:, 3 * _HID:4 * _HID]) * jnp.tanh(cb)
        hseq_ref[:, pl.ds(tb, 1), _HID:_ENC] = hb.reshape(_B, 1, _HID)
        return hf, cf, hb, cb

    def step16(i, carry):
        for j in range(64):
            carry = substep(64 * i + j, *carry)
        return carry

    zinit = jnp.zeros((_B, _HID), f32)
    lax.fori_loop(0, _T // 64, step16, (zinit, zinit, zinit, zinit))

    qvt = jnp.dot(wqt_ref[...], qpt_ref[...], preferred_element_type=f32)
    erow = lax.broadcasted_iota(jnp.int32, (_ENC, _NH), 0) // _DH
    hcol = lax.broadcasted_iota(jnp.int32, (_ENC, _NH), 1)
    qmat = jnp.where(erow == hcol, qvt, jnp.zeros((), f32))

    cp_k.wait()
    cp_v.wait()
    cp_o.wait()
    wk = wk_v[...]
    wv = wv_v[...]
    srows = []
    for b in range(_B):
        h1b = hseq_ref[b]
        kb = jnp.dot(h1b, wk, preferred_element_type=f32)
        vseq_ref[b] = jnp.dot(h1b, wv, preferred_element_type=f32)
        sbt = lax.dot_general(qmat, kb, (((0,), (1,)), ((), ())),
                              preferred_element_type=f32)
        srows.append(_SCALING * sbt)
    scores = jnp.concatenate(srows, axis=0)

    bits = lax.bitcast_convert_type(scores, jnp.int32)
    keys = jnp.where(bits >= 0, bits, jnp.bitwise_xor(bits, jnp.int32(0x7FFFFFFF)))
    cnt0 = jnp.sum((keys >= 0).astype(jnp.int32), axis=1, keepdims=True)
    big = cnt0 >= _KBUD
    lo = jnp.where(big, jnp.int32(0), jnp.int32(-2147483647 - 1))
    hi = jnp.where(big, jnp.int32(2147483647), jnp.int32(-1))

    def bstep(_, lh):
        blo, bhi = lh
        mid = bhi - ((bhi - blo) >> 1)
        cnt = jnp.sum((keys >= mid).astype(jnp.int32), axis=1, keepdims=True)
        ok = cnt >= _KBUD
        return jnp.where(ok, mid, blo), jnp.where(ok, bhi, mid - 1)

    lo, hi = lax.fori_loop(0, 32, bstep, (lo, hi))

    masked = jnp.where(keys >= lo, scores, jnp.full((), -1e9, f32))
    m = jnp.max(masked, axis=1, keepdims=True)
    e = jnp.exp(masked - m)
    probs = e / jnp.sum(e, axis=1, keepdims=True)

    hm = (lax.broadcasted_iota(jnp.int32, (_NH, _ENC), 1) // _DH
          == lax.broadcasted_iota(jnp.int32, (_NH, _ENC), 0)).astype(f32)
    prows = []
    zrows = []
    for b in range(_B):
        pb = probs[b * _NH:(b + 1) * _NH, :]
        pmat = jnp.dot(pb, vseq_ref[b], preferred_element_type=f32)
        prows.append(jnp.sum(pmat * hm, axis=0, keepdims=True))
        zrows.append(jnp.mean(pb, axis=0, keepdims=True))
    pooled = jnp.concatenate(prows, axis=0)
    h2 = jnp.dot(pooled, wo_v[...], preferred_element_type=f32)
    y_ref[...] = jax.nn.sigmoid(
        jnp.dot(h2, wout_ref[...], preferred_element_type=f32) + bout_ref[...])
    z_ref[...] = jnp.concatenate(zrows, axis=0) * maskf_ref[...]


def _fused_call(emb, wif, whf, bf, wib, whb, bb, wqt, qpt, wk, wv, wo, wout,
                bout, maskf):
    f32 = jnp.float32
    return pl.pallas_call(
        _fused_kernel,
        out_shape=[
            jax.ShapeDtypeStruct((_B, 1), f32),
            jax.ShapeDtypeStruct((_B, _T), f32),
        ],
        scratch_shapes=[
            pltpu.VMEM((_T * _B, _GH), f32),
            pltpu.VMEM((_T * _B, _GH), f32),
            pltpu.VMEM((_B, _T, _ENC), f32),
            pltpu.VMEM((_B, _T, _ENC), f32),
            pltpu.VMEM((_ENC, _ENC), f32),
            pltpu.VMEM((_ENC, _ENC), f32),
            pltpu.VMEM((_ENC, _ENC), f32),
            pltpu.SemaphoreType.DMA,
            pltpu.SemaphoreType.DMA,
            pltpu.SemaphoreType.DMA,
        ],
        in_specs=[pl.BlockSpec(memory_space=pltpu.VMEM)] * 9
        + [pl.BlockSpec(memory_space=pl.ANY)] * 3
        + [pl.BlockSpec(memory_space=pltpu.VMEM)] * 3,
        compiler_params=pltpu.CompilerParams(
            vmem_limit_bytes=128 * 1024 * 1024,
        ),
    )(emb, wif, whf, bf, wib, whb, bb, wqt, qpt, wk, wv, wo, wout, bout, maskf)


def kernel(x, z, mask, embed, Wi_f, Wh_f, b_f, Wi_b, Wh_b, b_b, q_pattern,
           Wq, Wk, Wv, Wo, W_out, b_out):
    del z
    idx = x.T.reshape(-1)
    emb = _sc_gather(embed, idx)
    y, z_out = _fused_call(
        emb, Wi_f, Wh_f, b_f.reshape(1, _GH), Wi_b, Wh_b, b_b.reshape(1, _GH),
        Wq.T, q_pattern.reshape(_ENC, 1), Wk, Wv, Wo, W_out,
        b_out.reshape(1, 1), mask.astype(jnp.float32))
    return (y, z_out)
